# baseline (speedup 1.0000x reference)
import functools
import math

import jax
import jax.numpy as jnp
from jax import lax
from jax.experimental import pallas as pl
from jax.experimental.pallas import tpu as pltpu
from jax.experimental.pallas import tpu_sc as plsc

EMB = 64
SCALE = math.sqrt(EMB)

NUM_CORES = 2
NUM_SUBCORES = 16
NUM_WORKERS = NUM_CORES * NUM_SUBCORES

B = 4096
T = 200
BPW = B // NUM_WORKERS
EG = EMB // 8
XP = T + 1
TP = BPW + 17
NBUF = 4


def _body(x_hbm, table_hbm, out_hbm, xs,
          idx0, idx1, idx2, idx3,
          rows0, rows1, rows2, rows3,
          tb0, tb1, tb2, tb3,
          xsem,
          gs0, gs1, gs2, gs3,
          ss0, ss1, ss2, ss3):
    wid = lax.axis_index("s") * NUM_CORES + lax.axis_index("c")
    pltpu.async_copy(
        x_hbm.at[pl.ds(wid * BPW, BPW), :], xs.at[:, pl.ds(0, T)], xsem
    ).wait()

    iota = lax.iota(jnp.int32, 16)
    idxs = (idx0, idx1, idx2, idx3)
    rows = (rows0, rows1, rows2, rows3)
    tbufs = (tb0, tb1, tb2, tb3)
    gsems = (gs0, gs1, gs2, gs3)
    ssems = (ss0, ss1, ss2, ss3)

    def prep_idx(t, p):
        for j in range(8):
            col = plsc.load_gather(
                xs, [iota + (j * 16), jnp.full((16,), 0, jnp.int32) + t]
            )
            idxs[p][pl.ds(j * 16, 16)] = col

    def start_gather(p):
        pltpu.async_copy(table_hbm.at[idxs[p]], rows[p], gsems[p])

    def wait_gather(p):
        pltpu.make_async_copy(
            table_hbm.at[idxs[p]], rows[p], gsems[p]
        ).wait()

    def transpose_scale(p):
        rv = rows[p]
        tbuf = tbufs[p]

        for k in range(4):
            e_vec = iota + (k * 16)

            @pl.loop(0, BPW, init_carry=jnp.zeros((16,), jnp.int32), unroll=8)
            def _r(r, rvec, k=k, e_vec=e_vec):
                vals = rv[r, pl.ds(k * 16, 16)]
                plsc.store_scatter(tbuf, [e_vec, rvec], vals * SCALE)
                return rvec + 1

    def start_stores(t, p):
        for eg in range(EG):
            pltpu.async_copy(
                tbufs[p].at[pl.ds(eg * 8, 8), pl.ds(0, BPW)],
                out_hbm.at[t, eg, wid],
                ssems[p],
            )

    def wait_stores(p):
        for eg in range(EG):
            pltpu.make_async_copy(
                tbufs[p].at[pl.ds(eg * 8, 8), pl.ds(0, BPW)],
                out_hbm.at[0, eg, wid],
                ssems[p],
            ).wait()

    for t in range(NBUF - 1):
        prep_idx(t, t)
        start_gather(t)

    @pl.loop(0, T // NBUF)
    def _quad(i):
        tbase = NBUF * i
        for p in range(NBUF):
            t = tbase + p
            pf = (p + NBUF - 1) % NBUF

            @pl.when(t + NBUF - 1 < T)
            def _():
                with jax.named_scope("prep_idx"):
                    prep_idx(t + NBUF - 1, pf)
                with jax.named_scope("start_gather"):
                    start_gather(pf)

            with jax.named_scope("wait_gather"):
                wait_gather(p)

            @pl.when(i > 0)
            def _():
                with jax.named_scope("wait_stores"):
                    wait_stores(p)

            with jax.named_scope("transpose"):
                transpose_scale(p)
            with jax.named_scope("start_stores"):
                start_stores(t, p)

    for p in range(NBUF):
        wait_stores(p)


def _make_kernel():
    mesh = plsc.VectorSubcoreMesh(core_axis_name="c", subcore_axis_name="s")
    return pl.kernel(
        _body,
        out_type=jax.ShapeDtypeStruct((T, EG, NUM_WORKERS, 8, BPW), jnp.float32),
        mesh=mesh,
        scratch_types=[
            pltpu.VMEM((BPW, XP), jnp.int32),
            pltpu.VMEM((BPW,), jnp.int32),
            pltpu.VMEM((BPW,), jnp.int32),
            pltpu.VMEM((BPW,), jnp.int32),
            pltpu.VMEM((BPW,), jnp.int32),
            pltpu.VMEM((BPW, EMB), jnp.float32),
            pltpu.VMEM((BPW, EMB), jnp.float32),
            pltpu.VMEM((BPW, EMB), jnp.float32),
            pltpu.VMEM((BPW, EMB), jnp.float32),
            pltpu.VMEM((EMB, TP), jnp.float32),
            pltpu.VMEM((EMB, TP), jnp.float32),
            pltpu.VMEM((EMB, TP), jnp.float32),
            pltpu.VMEM((EMB, TP), jnp.float32),
            pltpu.SemaphoreType.DMA,
            pltpu.SemaphoreType.DMA,
            pltpu.SemaphoreType.DMA,
            pltpu.SemaphoreType.DMA,
            pltpu.SemaphoreType.DMA,
            pltpu.SemaphoreType.DMA,
            pltpu.SemaphoreType.DMA,
            pltpu.SemaphoreType.DMA,
            pltpu.SemaphoreType.DMA,
        ],
        compiler_params=pltpu.CompilerParams(
            use_tc_tiling_on_sc=False, needs_layout_passes=False
        ),
    )


def kernel(x, table):
    out5 = _make_kernel()(x, table)
    a6 = jnp.transpose(out5, (0, 1, 3, 2, 4))
    r = jnp.reshape(a6, (T, EMB, B))
    return jnp.transpose(r, (2, 0, 1))


# speedup vs baseline: 2.9364x; 2.9364x over previous
import functools
import math

import jax
import jax.numpy as jnp
from jax import lax
from jax.experimental import pallas as pl
from jax.experimental.pallas import tpu as pltpu
from jax.experimental.pallas import tpu_sc as plsc

EMB = 64
SCALE = math.sqrt(EMB)

NUM_CORES = 2
NUM_SUBCORES = 16
NUM_WORKERS = NUM_CORES * NUM_SUBCORES

B = 4096
T = 200
BPW = B // NUM_WORKERS
EG = EMB // 8
XP = T + 1
TP = BPW + 17
NBUF = 4


def _body(x_hbm, table_hbm, out_hbm, xs,
          idx0, idx1, idx2, idx3,
          rows0, rows1, rows2, rows3,
          tb0, tb1, tb2, tb3,
          xsem,
          gs0, gs1, gs2, gs3,
          ss0, ss1, ss2, ss3):
    wid = lax.axis_index("s") * NUM_CORES + lax.axis_index("c")
    pltpu.async_copy(
        x_hbm.at[pl.ds(wid * BPW, BPW), :], xs.at[:, pl.ds(0, T)], xsem
    ).wait()

    iota = lax.iota(jnp.int32, 16)
    idxs = (idx0, idx1, idx2, idx3)
    rows = (rows0, rows1, rows2, rows3)
    tbufs = (tb0, tb1, tb2, tb3)
    gsems = (gs0, gs1, gs2, gs3)
    ssems = (ss0, ss1, ss2, ss3)

    def prep_idx(t, p):
        for j in range(8):
            col = plsc.load_gather(
                xs, [iota + (j * 16), jnp.full((16,), 0, jnp.int32) + t]
            )
            idxs[p][pl.ds(j * 16, 16)] = col

    def start_gather(p):
        pltpu.async_copy(table_hbm.at[idxs[p]], rows[p], gsems[p])

    def wait_gather(p):
        pltpu.make_async_copy(
            table_hbm.at[idxs[p]], rows[p], gsems[p]
        ).wait()

    def transpose_scale(p):
        rv = rows[p]
        tbuf = tbufs[p]

        for k in range(4):
            e_vec = iota + (k * 16)

            @plsc.parallel_loop(0, BPW, unroll=8, carry=jnp.zeros((16,), jnp.int32))
            def _r(r, rvec, k=k, e_vec=e_vec):
                vals = rv[r, pl.ds(k * 16, 16)]
                plsc.store_scatter(tbuf, [e_vec, rvec], vals * SCALE)
                return rvec + 1

    def start_stores(t, p):
        for eg in range(EG):
            pltpu.async_copy(
                tbufs[p].at[pl.ds(eg * 8, 8), pl.ds(0, BPW)],
                out_hbm.at[t, eg, wid],
                ssems[p],
            )

    def wait_stores(p):
        for eg in range(EG):
            pltpu.make_async_copy(
                tbufs[p].at[pl.ds(eg * 8, 8), pl.ds(0, BPW)],
                out_hbm.at[0, eg, wid],
                ssems[p],
            ).wait()

    for t in range(NBUF - 1):
        prep_idx(t, t)
        start_gather(t)

    @pl.loop(0, T // NBUF)
    def _quad(i):
        tbase = NBUF * i
        for p in range(NBUF):
            t = tbase + p
            pf = (p + NBUF - 1) % NBUF

            @pl.when(t + NBUF - 1 < T)
            def _():
                with jax.named_scope("prep_idx"):
                    prep_idx(t + NBUF - 1, pf)
                with jax.named_scope("start_gather"):
                    start_gather(pf)

            with jax.named_scope("wait_gather"):
                wait_gather(p)

            @pl.when(i > 0)
            def _():
                with jax.named_scope("wait_stores"):
                    wait_stores(p)

            with jax.named_scope("transpose"):
                transpose_scale(p)
            with jax.named_scope("start_stores"):
                start_stores(t, p)

    for p in range(NBUF):
        wait_stores(p)


def _make_kernel():
    mesh = plsc.VectorSubcoreMesh(core_axis_name="c", subcore_axis_name="s")
    return pl.kernel(
        _body,
        out_type=jax.ShapeDtypeStruct((T, EG, NUM_WORKERS, 8, BPW), jnp.float32),
        mesh=mesh,
        scratch_types=[
            pltpu.VMEM((BPW, XP), jnp.int32),
            pltpu.VMEM((BPW,), jnp.int32),
            pltpu.VMEM((BPW,), jnp.int32),
            pltpu.VMEM((BPW,), jnp.int32),
            pltpu.VMEM((BPW,), jnp.int32),
            pltpu.VMEM((BPW, EMB), jnp.float32),
            pltpu.VMEM((BPW, EMB), jnp.float32),
            pltpu.VMEM((BPW, EMB), jnp.float32),
            pltpu.VMEM((BPW, EMB), jnp.float32),
            pltpu.VMEM((EMB, TP), jnp.float32),
            pltpu.VMEM((EMB, TP), jnp.float32),
            pltpu.VMEM((EMB, TP), jnp.float32),
            pltpu.VMEM((EMB, TP), jnp.float32),
            pltpu.SemaphoreType.DMA,
            pltpu.SemaphoreType.DMA,
            pltpu.SemaphoreType.DMA,
            pltpu.SemaphoreType.DMA,
            pltpu.SemaphoreType.DMA,
            pltpu.SemaphoreType.DMA,
            pltpu.SemaphoreType.DMA,
            pltpu.SemaphoreType.DMA,
            pltpu.SemaphoreType.DMA,
        ],
        compiler_params=pltpu.CompilerParams(
            use_tc_tiling_on_sc=False, needs_layout_passes=False
        ),
    )


def kernel(x, table):
    out5 = _make_kernel()(x, table)
    a6 = jnp.transpose(out5, (0, 1, 3, 2, 4))
    r = jnp.reshape(a6, (T, EMB, B))
    return jnp.transpose(r, (2, 0, 1))
